# baseline (speedup 1.0000x reference)
import jax
import jax.numpy as jnp
from jax import lax
from jax.experimental import pallas as pl
from jax.experimental.pallas import tpu as pltpu
from jax.experimental.pallas import tpu_sc as plsc

NC = 2
NS = 16
BLK = 128
NBUF = 4
MR = 8
PF = 2


def _mm_body(x_ref, w_ref, o_ref):
    o_ref[0] = jnp.dot(x_ref[...], w_ref[0],
                       preferred_element_type=jnp.float32)


def _matmul(x, Wsp, npd):
    n, d_in = x.shape
    dh = Wsp.shape[-1]
    bn = 1000
    return pl.pallas_call(
        _mm_body,
        grid=(NC, n // bn),
        in_specs=[pl.BlockSpec((bn, d_in), lambda c, i: (i, 0)),
                  pl.BlockSpec((1, d_in, dh), lambda c, i: (c, 0, 0))],
        out_specs=pl.BlockSpec((1, bn, dh), lambda c, i: (c, i, 0)),
        out_shape=jax.ShapeDtypeStruct((NC, npd, dh), jnp.float32),
    )(x, Wsp)


def _fin_body(p_ref, b_ref, o_ref):
    o_ref[...] = jnp.maximum(
        jnp.concatenate([p_ref[0], p_ref[1]], axis=-1) + b_ref[...], 0.0)


def _finish(partials, b, n):
    dh = partials.shape[-1]
    bn = 1000
    b2 = b.reshape(1, 2 * dh)
    return pl.pallas_call(
        _fin_body,
        grid=(n // bn,),
        in_specs=[pl.BlockSpec((2, bn, dh), lambda i: (0, i, 0)),
                  pl.BlockSpec((1, 2 * dh), lambda i: (0, 0))],
        out_specs=pl.BlockSpec((bn, 2 * dh), lambda i: (i, 0)),
        out_shape=jax.ShapeDtypeStruct((n, 2 * dh), jnp.float32),
    )(partials, b2)


def _sc_propagate(sup, meta, nb, npd):
    _, _, dh = sup.shape
    rchunk = BLK
    rpt = npd // NS
    nq = dh // 16
    assert nb % MR == 0

    def body(sup_hbm, meta_hbm, zin_hbm, out_hbm, meta_v, *rest):
        gbufs = rest[:NBUF]
        acc, gs, ss, ms = rest[NBUF:]
        gb0 = gbufs[0]
        cid = lax.axis_index("c")
        sid = lax.axis_index("s")
        sup_c = sup_hbm.at[cid]

        r0 = sid * rpt
        pltpu.sync_copy(zin_hbm, gb0)
        for k in range(rpt // rchunk):
            pltpu.sync_copy(gb0, acc.at[pl.ds(r0 + k * rchunk, rchunk)])

        def mstart(mf, blk):
            pltpu.async_copy(meta_hbm.at[sid, blk], meta_v.at[mf], ms.at[mf])

        def mwait(mf):
            pltpu.make_async_copy(
                meta_hbm.at[sid, 0], meta_v.at[mf], ms.at[mf]).wait()

        def gstart(bf, mf):
            pltpu.async_copy(
                sup_c.at[meta_v.at[mf, 0]], gbufs[bf], gs.at[bf])

        def gwait(bf):
            pltpu.make_async_copy(
                sup_c.at[meta_v.at[0, 0]], gbufs[bf], gs.at[bf]).wait()

        def sstart(bf, mf):
            pltpu.async_copy(
                gbufs[bf], acc.at[meta_v.at[mf, 1]], ss.at[bf], add=True)

        def swait(bf):
            pltpu.make_async_copy(
                gbufs[bf], acc.at[meta_v.at[0, 1]], ss.at[bf]).wait()

        for m in range(MR):
            mstart(m, m)
        plsc.subcore_barrier()
        for m in range(PF):
            mwait(m)
            gstart(m, m)

        def loop_body(j16, carry):
            for r in range(MR):
                j = j16 * MR + r
                b = r % NBUF
                gb = gbufs[b]
                gwait(b)

                def grp_body(t, c):
                    w16 = plsc.bitcast(
                        meta_v[r, 2, pl.ds(t * 16, 16)], jnp.float32)
                    for i in range(16):
                        e = t * 16 + i
                        wv = w16[i]
                        for q in range(nq):
                            sl = pl.ds(q * 16, 16)
                            gb[e, sl] = gb[e, sl] * wv
                    return c
                lax.fori_loop(0, BLK // 16, grp_body, 0)

                sstart(b, r)

                def tail():
                    swait((b + PF) % NBUF)
                    mstart((r + 3 * PF) % MR, j + 3 * PF)
                if r < PF:
                    pl.when(j16 > 0)(tail)
                else:
                    tail()

                mwait((r + PF) % MR)
                gstart((b + PF) % NBUF, (r + PF) % MR)
            return carry

        lax.fori_loop(0, nb // MR, loop_body, 0)

        for k in range(PF):
            swait((nb - PF + k) % NBUF)
            gwait((nb + k) % NBUF)
        for k in range(PF, 3 * PF):
            mwait((nb + k) % MR)
        plsc.subcore_barrier()

        for k in range(rpt // rchunk):
            rr = r0 + k * rchunk
            pltpu.sync_copy(acc.at[pl.ds(rr, rchunk)], gb0)
            pltpu.sync_copy(gb0, out_hbm.at[cid, pl.ds(rr, rchunk)])

    zin = jnp.zeros((BLK, dh), jnp.float32)
    run = pl.kernel(
        body,
        out_type=jax.ShapeDtypeStruct((NC, npd, dh), jnp.float32),
        mesh=plsc.VectorSubcoreMesh(core_axis_name="c", subcore_axis_name="s"),
        compiler_params=pltpu.CompilerParams(use_tc_tiling_on_sc=False,
                                             needs_layout_passes=False,
                                             disable_bounds_checks=True,
                                             skip_device_barrier=True),
        scratch_types=(
            [pltpu.VMEM((MR, 3, BLK), jnp.int32)]
            + [pltpu.VMEM((BLK, dh), jnp.float32) for _ in range(NBUF)]
            + [pltpu.VMEM_SHARED((npd, dh), jnp.float32),
               pltpu.SemaphoreType.DMA((NBUF,)),
               pltpu.SemaphoreType.DMA((NBUF,)),
               pltpu.SemaphoreType.DMA((MR,))]
        ),
    )
    return run(sup, meta, zin)


def kernel(x, edge_index, edge_weight, W, b):
    n, d_in = x.shape
    d = W.shape[1]
    dh = d // NC
    e = edge_weight.shape[0]

    src = edge_index[0].astype(jnp.int32)
    dst = edge_index[1].astype(jnp.int32)
    wbits = lax.bitcast_convert_type(edge_weight.astype(jnp.float32),
                                     jnp.int32)

    nb = -(-e // (NS * BLK))
    nb = -(-nb // MR) * MR
    ep = NS * nb * BLK
    pad = ep - e
    src = jnp.concatenate([src, jnp.zeros((pad,), jnp.int32)])
    dst = jnp.concatenate([dst, jnp.zeros((pad,), jnp.int32)])
    wbits = jnp.concatenate([wbits, jnp.zeros((pad,), jnp.int32)])
    meta = jnp.stack([src.reshape(NS, nb, BLK),
                      dst.reshape(NS, nb, BLK),
                      wbits.reshape(NS, nb, BLK)], axis=2)
    meta = jnp.pad(meta, ((0, 0), (0, MR), (0, 0), (0, 0)))

    rpt = -(-(-(-n // NS)) // BLK) * BLK
    npd = NS * rpt

    Wsp = W.reshape(d_in, NC, dh).transpose(1, 0, 2)
    sup = _matmul(x, Wsp, npd)
    partials = _sc_propagate(sup, meta, nb, npd)
    return _finish(partials, b, n)


# speedup vs baseline: 2.1517x; 2.1517x over previous
import jax
import jax.numpy as jnp
from jax import lax
from jax.experimental import pallas as pl
from jax.experimental.pallas import tpu as pltpu
from jax.experimental.pallas import tpu_sc as plsc

NC = 2
NS = 16
BLK = 128
NBUF = 4
MR = 8
PF = 2


def _mm_body(x_ref, w_ref, o_ref):
    o_ref[0] = jnp.dot(x_ref[...], w_ref[0],
                       preferred_element_type=jnp.float32)


def _matmul(x, Wsp, npd):
    n, d_in = x.shape
    dh = Wsp.shape[-1]
    bn = 1000
    return pl.pallas_call(
        _mm_body,
        grid=(NC, n // bn),
        in_specs=[pl.BlockSpec((bn, d_in), lambda c, i: (i, 0)),
                  pl.BlockSpec((1, d_in, dh), lambda c, i: (c, 0, 0))],
        out_specs=pl.BlockSpec((1, bn, dh), lambda c, i: (c, i, 0)),
        out_shape=jax.ShapeDtypeStruct((NC, npd, dh), jnp.float32),
    )(x, Wsp)


def _fin_body(p_ref, b_ref, o_ref):
    o_ref[...] = jnp.maximum(
        jnp.concatenate([p_ref[0], p_ref[1]], axis=-1) + b_ref[...], 0.0)


def _finish(partials, b, n):
    dh = partials.shape[-1]
    bn = 1000
    b2 = b.reshape(1, 2 * dh)
    return pl.pallas_call(
        _fin_body,
        grid=(n // bn,),
        in_specs=[pl.BlockSpec((2, bn, dh), lambda i: (0, i, 0)),
                  pl.BlockSpec((1, 2 * dh), lambda i: (0, 0))],
        out_specs=pl.BlockSpec((bn, 2 * dh), lambda i: (i, 0)),
        out_shape=jax.ShapeDtypeStruct((n, 2 * dh), jnp.float32),
    )(partials, b2)


def _sc_propagate(sup, meta, nb, npd):
    _, _, dh = sup.shape
    rchunk = BLK
    rpt = npd // NS
    nq = dh // 16
    assert nb % MR == 0

    def body(sup_hbm, meta_hbm, zin_hbm, out_hbm, meta_v, *rest):
        gbufs = rest[:NBUF]
        acc, sup_s, gs, ss, ms = rest[NBUF:]
        gb0 = gbufs[0]
        cid = lax.axis_index("c")
        sid = lax.axis_index("s")
        sup_c = sup_s

        r0 = sid * rpt
        pltpu.sync_copy(sup_hbm.at[cid, pl.ds(r0, rpt)],
                        sup_s.at[pl.ds(r0, rpt)])
        pltpu.sync_copy(zin_hbm, gb0)
        for k in range(rpt // rchunk):
            pltpu.sync_copy(gb0, acc.at[pl.ds(r0 + k * rchunk, rchunk)])

        def mstart(mf, blk):
            pltpu.async_copy(meta_hbm.at[sid, blk], meta_v.at[mf], ms.at[mf])

        def mwait(mf):
            pltpu.make_async_copy(
                meta_hbm.at[sid, 0], meta_v.at[mf], ms.at[mf]).wait()

        def gstart(bf, mf):
            pltpu.async_copy(
                sup_c.at[meta_v.at[mf, 0]], gbufs[bf], gs.at[bf])

        def gwait(bf):
            pltpu.make_async_copy(
                sup_c.at[meta_v.at[0, 0]], gbufs[bf], gs.at[bf]).wait()

        def sstart(bf, mf):
            pltpu.async_copy(
                gbufs[bf], acc.at[meta_v.at[mf, 1]], ss.at[bf], add=True)

        def swait(bf):
            pltpu.make_async_copy(
                gbufs[bf], acc.at[meta_v.at[0, 1]], ss.at[bf]).wait()

        for m in range(MR):
            mstart(m, m)
        plsc.subcore_barrier()
        for m in range(PF):
            mwait(m)
            gstart(m, m)

        def loop_body(j16, carry):
            for r in range(MR):
                j = j16 * MR + r
                b = r % NBUF
                gb = gbufs[b]
                gwait(b)

                def grp_body(t, c):
                    w16 = plsc.bitcast(
                        meta_v[r, 2, pl.ds(t * 16, 16)], jnp.float32)
                    for i in range(16):
                        e = t * 16 + i
                        wv = w16[i]
                        for q in range(nq):
                            sl = pl.ds(q * 16, 16)
                            gb[e, sl] = gb[e, sl] * wv
                    return c
                lax.fori_loop(0, BLK // 16, grp_body, 0)

                sstart(b, r)

                def tail():
                    swait((b + PF) % NBUF)
                    mstart((r + 3 * PF) % MR, j + 3 * PF)
                if r < PF:
                    pl.when(j16 > 0)(tail)
                else:
                    tail()

                mwait((r + PF) % MR)
                gstart((b + PF) % NBUF, (r + PF) % MR)
            return carry

        lax.fori_loop(0, nb // MR, loop_body, 0)

        for k in range(PF):
            swait((nb - PF + k) % NBUF)
            gwait((nb + k) % NBUF)
        for k in range(PF, 3 * PF):
            mwait((nb + k) % MR)
        plsc.subcore_barrier()

        for k in range(rpt // rchunk):
            rr = r0 + k * rchunk
            pltpu.sync_copy(acc.at[pl.ds(rr, rchunk)], gb0)
            pltpu.sync_copy(gb0, out_hbm.at[cid, pl.ds(rr, rchunk)])

    zin = jnp.zeros((BLK, dh), jnp.float32)
    run = pl.kernel(
        body,
        out_type=jax.ShapeDtypeStruct((NC, npd, dh), jnp.float32),
        mesh=plsc.VectorSubcoreMesh(core_axis_name="c", subcore_axis_name="s"),
        compiler_params=pltpu.CompilerParams(use_tc_tiling_on_sc=False,
                                             needs_layout_passes=False,
                                             disable_bounds_checks=True,
                                             skip_device_barrier=True),
        scratch_types=(
            [pltpu.VMEM((MR, 3, BLK), jnp.int32)]
            + [pltpu.VMEM((BLK, dh), jnp.float32) for _ in range(NBUF)]
            + [pltpu.VMEM_SHARED((npd, dh), jnp.float32),
               pltpu.VMEM_SHARED((npd, dh), jnp.float32),
               pltpu.SemaphoreType.DMA((NBUF,)),
               pltpu.SemaphoreType.DMA((NBUF,)),
               pltpu.SemaphoreType.DMA((MR,))]
        ),
    )
    return run(sup, meta, zin)


def kernel(x, edge_index, edge_weight, W, b):
    n, d_in = x.shape
    d = W.shape[1]
    dh = d // NC
    e = edge_weight.shape[0]

    src = edge_index[0].astype(jnp.int32)
    dst = edge_index[1].astype(jnp.int32)
    wbits = lax.bitcast_convert_type(edge_weight.astype(jnp.float32),
                                     jnp.int32)

    nb = -(-e // (NS * BLK))
    nb = -(-nb // MR) * MR
    ep = NS * nb * BLK
    pad = ep - e
    src = jnp.concatenate([src, jnp.zeros((pad,), jnp.int32)])
    dst = jnp.concatenate([dst, jnp.zeros((pad,), jnp.int32)])
    wbits = jnp.concatenate([wbits, jnp.zeros((pad,), jnp.int32)])
    meta = jnp.stack([src.reshape(NS, nb, BLK),
                      dst.reshape(NS, nb, BLK),
                      wbits.reshape(NS, nb, BLK)], axis=2)
    meta = jnp.pad(meta, ((0, 0), (0, MR), (0, 0), (0, 0)))

    rpt = -(-(-(-n // NS)) // BLK) * BLK
    npd = NS * rpt

    Wsp = W.reshape(d_in, NC, dh).transpose(1, 0, 2)
    sup = _matmul(x, Wsp, npd)
    partials = _sc_propagate(sup, meta, nb, npd)
    return _finish(partials, b, n)
